# Initial kernel scaffold; baseline (speedup 1.0000x reference)
import jax
import jax.numpy as jnp
from jax.experimental import pallas as pl


def kernel(input, W1, b1, W2, b2, W3, b3, W4, b4, W5, b5, W6, b6, W7, b7):
    raise NotImplementedError("write your pallas kernel here")


# speedup vs baseline: 4.6121x; 4.6121x over previous
import functools

import jax
import jax.numpy as jnp
from jax import lax
from jax.experimental import pallas as pl
from jax.experimental.pallas import tpu as pltpu
from jax.experimental.pallas import tpu_sc as plsc

ID_LEN = 64
DIN = 128
NB = 4
NA = 512
NTOK = NB * NA
NE = 64
TILE = 128
NBLK = NTOK // TILE
NT = NBLK + NE
NPAD = NT * TILE
ACTS = 64
NW = 32
RPW = NTOK // NW


def _routing_body(inp_ref, pos_ref, texp_ref, ids_s, rank_s, cnt_s):
    r = pl.program_id(0)

    @pl.when(r == 0)
    def _():
        cnt_s[...] = jnp.zeros((NE,), jnp.float32)

    @pl.when(r < NBLK)
    def _():
        blk = inp_ref[...]
        tail = blk[:, DIN:]
        m = jnp.max(tail, axis=1, keepdims=True)
        iota_e = lax.broadcasted_iota(jnp.int32, (TILE, ID_LEN), 1)
        ids_b = jnp.min(jnp.where(tail == m, iota_e, ID_LEN), axis=1)
        onehot = (iota_e == ids_b[:, None]).astype(jnp.float32)
        row_i = lax.broadcasted_iota(jnp.int32, (TILE, TILE), 0)
        col_i = lax.broadcasted_iota(jnp.int32, (TILE, TILE), 1)
        lstrict = (row_i > col_i).astype(jnp.float32)
        rel = jnp.sum(jnp.dot(lstrict, onehot,
                              preferred_element_type=jnp.float32) * onehot, axis=1)
        cnt = cnt_s[...]
        rank = jnp.sum(onehot * cnt[None, :], axis=1) + rel
        ids_s[pl.ds(r, 1), :] = ids_b[None, :]
        rank_s[pl.ds(r, 1), :] = rank.astype(jnp.int32)[None, :]
        cnt_s[...] = cnt + jnp.sum(onehot, axis=0)

    @pl.when(r == NBLK)
    def _():
        cnt = cnt_s[...]
        tiles_e = jnp.floor((cnt + (TILE - 1)) * (1.0 / TILE))
        e_row = lax.broadcasted_iota(jnp.int32, (NE, NE), 0)
        e_col = lax.broadcasted_iota(jnp.int32, (NE, NE), 1)
        l64 = (e_row > e_col).astype(jnp.float32)
        start_t = jnp.dot(l64, tiles_e, preferred_element_type=jnp.float32)
        offsets = start_t * TILE
        ids3 = ids_s[...][:, :, None]
        iota_e3 = lax.broadcasted_iota(jnp.int32, (NBLK, TILE, NE), 2)
        onehot3 = (iota_e3 == ids3).astype(jnp.float32)
        off2 = jnp.sum(onehot3 * offsets[None, None, :], axis=2)
        pos = off2 + rank_s[...].astype(jnp.float32)
        pos_ref[...] = pos.astype(jnp.int32)
        t_row = lax.broadcasted_iota(jnp.int32, (NT, NE), 0).astype(jnp.float32)
        le = (start_t[None, :] <= t_row).astype(jnp.float32)
        texp_ref[...] = (jnp.sum(le, axis=1) - 1.0).astype(jnp.int32)


def _routing(inp2d):
    return pl.pallas_call(
        _routing_body,
        grid=(NBLK + 1,),
        in_specs=[pl.BlockSpec((TILE, DIN + ID_LEN),
                               lambda r: (jnp.minimum(r, NBLK - 1), 0))],
        out_specs=[pl.BlockSpec((NBLK, TILE), lambda r: (0, 0)),
                   pl.BlockSpec((NT,), lambda r: (0,))],
        out_shape=[jax.ShapeDtypeStruct((NBLK, TILE), jnp.int32),
                   jax.ShapeDtypeStruct((NT,), jnp.int32)],
        scratch_shapes=[pltpu.VMEM((NBLK, TILE), jnp.int32),
                        pltpu.VMEM((NBLK, TILE), jnp.int32),
                        pltpu.VMEM((NE,), jnp.float32)],
    )(inp2d)


@functools.lru_cache(maxsize=None)
def _sc_kernels():
    mesh = plsc.VectorSubcoreMesh(core_axis_name="c", subcore_axis_name="s")

    @functools.partial(
        pl.kernel,
        out_type=jax.ShapeDtypeStruct((NPAD, DIN), jnp.float32),
        mesh=mesh,
        scratch_types=[pltpu.VMEM((RPW,), jnp.int32),
                       pltpu.VMEM((RPW, DIN), jnp.float32),
                       pltpu.SemaphoreType.DMA],
    )
    def sc_scatter(x_hbm, pos_hbm, xpad_hbm, idx_v, rows_v, sem):
        wid = lax.axis_index("s") * mesh.num_cores + lax.axis_index("c")
        base = wid * RPW
        pltpu.sync_copy(x_hbm.at[pl.ds(base, RPW)], rows_v)
        pltpu.sync_copy(pos_hbm.at[pl.ds(base, RPW)], idx_v)
        pltpu.async_copy(rows_v, xpad_hbm.at[idx_v], sem).wait()

    @functools.partial(
        pl.kernel,
        out_type=jax.ShapeDtypeStruct((NTOK, DIN), jnp.float32),
        mesh=mesh,
        scratch_types=[pltpu.VMEM((RPW,), jnp.int32),
                       pltpu.VMEM((RPW, DIN), jnp.float32),
                       pltpu.SemaphoreType.DMA],
    )
    def sc_gather(ypad_hbm, pos_hbm, out_hbm, idx_v, rows_v, sem):
        wid = lax.axis_index("s") * mesh.num_cores + lax.axis_index("c")
        base = wid * RPW
        pltpu.sync_copy(pos_hbm.at[pl.ds(base, RPW)], idx_v)
        pltpu.async_copy(ypad_hbm.at[idx_v], rows_v, sem).wait()
        pltpu.sync_copy(rows_v, out_hbm.at[pl.ds(base, RPW)])

    return sc_scatter, sc_gather


def _mlp_body(te_ref, x_ref, w1, b1, w2, b2, w3, b3, w4, b4, w5, b5,
              w6, b6, w7, b7, y_ref):
    def lay(x, wr, br, relu):
        w = wr[0]
        y = lax.dot_general(x, w, (((1,), (1,)), ((), ())),
                            preferred_element_type=jnp.float32) + br[0]
        return jnp.maximum(y, 0.0) if relu else y

    h = lay(x_ref[...], w1, b1, True)
    h = lay(h, w2, b2, True)
    h = lay(h, w3, b3, False)
    h = lay(h, w4, b4, True)
    h = lay(h, w5, b5, True)
    h = lay(h, w6, b6, False)
    y = lay(h, w7, b7, False)
    y_ref[...] = jnp.concatenate(
        [y, jnp.zeros((TILE, DIN - ACTS), jnp.float32)], axis=1)


def _mlp(texp, xpad, W1, b1, W2, b2, W3, b3, W4, b4, W5, b5, W6, b6, W7, b7):
    def wspec(out_dim):
        return pl.BlockSpec((1, out_dim, DIN), lambda i, te: (te[i], 0, 0))

    def bspec(out_dim):
        return pl.BlockSpec((1, 1, out_dim), lambda i, te: (te[i], 0, 0))

    grid_spec = pltpu.PrefetchScalarGridSpec(
        num_scalar_prefetch=1,
        grid=(NT,),
        in_specs=[pl.BlockSpec((TILE, DIN), lambda i, te: (i, 0)),
                  wspec(128), bspec(128), wspec(128), bspec(128),
                  wspec(128), bspec(128), wspec(128), bspec(128),
                  wspec(128), bspec(128), wspec(128), bspec(128),
                  wspec(ACTS), bspec(ACTS)],
        out_specs=pl.BlockSpec((TILE, DIN), lambda i, te: (i, 0)),
    )
    return pl.pallas_call(
        _mlp_body,
        grid_spec=grid_spec,
        out_shape=jax.ShapeDtypeStruct((NPAD, DIN), jnp.float32),
    )(texp, xpad,
      W1, b1.reshape(NE, 1, -1), W2, b2.reshape(NE, 1, -1),
      W3, b3.reshape(NE, 1, -1), W4, b4.reshape(NE, 1, -1),
      W5, b5.reshape(NE, 1, -1), W6, b6.reshape(NE, 1, -1),
      W7, b7.reshape(NE, 1, -1))


def kernel(input, W1, b1, W2, b2, W3, b3, W4, b4, W5, b5, W6, b6, W7, b7):
    inp2d = input.reshape(NTOK, DIN + ID_LEN)
    sc_scatter, sc_gather = _sc_kernels()
    pos2d, texp = _routing(inp2d)
    pos = pos2d.reshape(NTOK)
    x = inp2d[:, :DIN]
    xpad = sc_scatter(x, pos)
    ypad = _mlp(texp, xpad, W1, b1, W2, b2, W3, b3, W4, b4, W5, b5,
                W6, b6, W7, b7)
    out = sc_gather(ypad, pos)
    return out[:, :ACTS].reshape(NB, NA, ACTS)


# speedup vs baseline: 5.4850x; 1.1893x over previous
import functools

import jax
import jax.numpy as jnp
from jax import lax
from jax.experimental import pallas as pl
from jax.experimental.pallas import tpu as pltpu
from jax.experimental.pallas import tpu_sc as plsc

ID_LEN = 64
DIN = 128
NB = 4
NA = 512
NTOK = NB * NA
NE = 64
TILE = 128
NBLK = NTOK // TILE
NT = NBLK + NE
NPAD = NT * TILE
ACTS = 64
NW = 32
RPW = NTOK // NW


def _routing_body(inp_ref, pos_ref, texp_ref):
    row_i = lax.broadcasted_iota(jnp.int32, (TILE, TILE), 0)
    col_i = lax.broadcasted_iota(jnp.int32, (TILE, TILE), 1)
    lstrict = (row_i > col_i).astype(jnp.float32)
    iota_e = lax.broadcasted_iota(jnp.int32, (TILE, ID_LEN), 1)

    onehots = []
    ranks = []
    cnt_run = jnp.zeros((1, NE), jnp.float32)
    for b in range(NBLK):
        tail = inp_ref[b][:, DIN:]
        m = jnp.max(tail, axis=1, keepdims=True)
        ids_b = jnp.min(jnp.where(tail == m, iota_e, ID_LEN), axis=1)
        onehot = (iota_e == ids_b[:, None]).astype(jnp.float32)
        rel = jnp.sum(jnp.dot(lstrict, onehot,
                              preferred_element_type=jnp.float32) * onehot, axis=1)
        rank = jnp.sum(onehot * cnt_run, axis=1) + rel
        cnt_run = cnt_run + jnp.sum(onehot, axis=0, keepdims=True)
        onehots.append(onehot)
        ranks.append(rank)

    cnt = cnt_run[0]
    tiles_e = jnp.floor((cnt + (TILE - 1)) * (1.0 / TILE))
    e_row = lax.broadcasted_iota(jnp.int32, (NE, NE), 0)
    e_col = lax.broadcasted_iota(jnp.int32, (NE, NE), 1)
    l64 = (e_row > e_col).astype(jnp.float32)
    start_t = jnp.dot(l64, tiles_e, preferred_element_type=jnp.float32)
    offsets = start_t * TILE
    for b in range(NBLK):
        pos_b = jnp.sum(onehots[b] * offsets[None, :], axis=1) + ranks[b]
        pos_ref[pl.ds(b, 1), :] = pos_b.astype(jnp.int32)[None, :]
    t_row = lax.broadcasted_iota(jnp.int32, (NT, NE), 0).astype(jnp.float32)
    le = (start_t[None, :] <= t_row).astype(jnp.float32)
    texp_ref[...] = (jnp.sum(le, axis=1) - 1.0).astype(jnp.int32)


def _routing(inp3d):
    return pl.pallas_call(
        _routing_body,
        out_shape=[jax.ShapeDtypeStruct((NBLK, TILE), jnp.int32),
                   jax.ShapeDtypeStruct((NT,), jnp.int32)],
    )(inp3d)


@functools.lru_cache(maxsize=None)
def _sc_kernels():
    mesh = plsc.VectorSubcoreMesh(core_axis_name="c", subcore_axis_name="s")

    @functools.partial(
        pl.kernel,
        out_type=jax.ShapeDtypeStruct((NPAD, DIN), jnp.float32),
        mesh=mesh,
        scratch_types=[pltpu.VMEM((RPW,), jnp.int32),
                       pltpu.VMEM((RPW, DIN), jnp.float32),
                       pltpu.SemaphoreType.DMA],
    )
    def sc_scatter(x_hbm, pos_hbm, xpad_hbm, idx_v, rows_v, sem):
        wid = lax.axis_index("s") * mesh.num_cores + lax.axis_index("c")
        base = wid * RPW
        pltpu.sync_copy(x_hbm.at[pl.ds(base, RPW)], rows_v)
        pltpu.sync_copy(pos_hbm.at[pl.ds(base, RPW)], idx_v)
        pltpu.async_copy(rows_v, xpad_hbm.at[idx_v], sem).wait()

    @functools.partial(
        pl.kernel,
        out_type=jax.ShapeDtypeStruct((NTOK, DIN), jnp.float32),
        mesh=mesh,
        scratch_types=[pltpu.VMEM((RPW,), jnp.int32),
                       pltpu.VMEM((RPW, DIN), jnp.float32),
                       pltpu.SemaphoreType.DMA],
    )
    def sc_gather(ypad_hbm, pos_hbm, out_hbm, idx_v, rows_v, sem):
        wid = lax.axis_index("s") * mesh.num_cores + lax.axis_index("c")
        base = wid * RPW
        pltpu.sync_copy(pos_hbm.at[pl.ds(base, RPW)], idx_v)
        pltpu.async_copy(ypad_hbm.at[idx_v], rows_v, sem).wait()
        pltpu.sync_copy(rows_v, out_hbm.at[pl.ds(base, RPW)])

    return sc_scatter, sc_gather


KT = 8
NSTEP = NT // KT


def _mlp_body(te_ref, x_ref, *args):
    wrefs, y_ref = args[:-1], args[-1]
    for j in range(KT):
        w1, b1, w2, b2, w3, b3, w4, b4, w5, b5, w6, b6, w7, b7 = \
            wrefs[j * 14:(j + 1) * 14]

        def lay(x, wr, br, relu):
            w = wr[0]
            y = lax.dot_general(x, w, (((1,), (1,)), ((), ())),
                                preferred_element_type=jnp.float32) + br[0]
            return jnp.maximum(y, 0.0) if relu else y

        h = lay(x_ref[pl.ds(j * TILE, TILE), :], w1, b1, True)
        h = lay(h, w2, b2, True)
        h = lay(h, w3, b3, False)
        h = lay(h, w4, b4, True)
        h = lay(h, w5, b5, True)
        h = lay(h, w6, b6, False)
        y = lay(h, w7, b7, False)
        y_ref[pl.ds(j * TILE, TILE), :] = jnp.concatenate(
            [y, jnp.zeros((TILE, DIN - ACTS), jnp.float32)], axis=1)


def _mlp(texp, xpad, W1, b1, W2, b2, W3, b3, W4, b4, W5, b5, W6, b6, W7, b7):
    def wspec(out_dim, j):
        return pl.BlockSpec((1, out_dim, DIN),
                            lambda i, te, j=j: (te[KT * i + j], 0, 0))

    def bspec(out_dim, j):
        return pl.BlockSpec((1, 1, out_dim),
                            lambda i, te, j=j: (te[KT * i + j], 0, 0))

    in_specs = [pl.BlockSpec((KT * TILE, DIN), lambda i, te: (i, 0))]
    operands = []
    ws = [(W1, b1), (W2, b2), (W3, b3), (W4, b4), (W5, b5), (W6, b6), (W7, b7)]
    for j in range(KT):
        for (W, b) in ws:
            in_specs.append(wspec(W.shape[1], j))
            in_specs.append(bspec(b.shape[1], j))
            operands.append(W)
            operands.append(b.reshape(NE, 1, -1))

    grid_spec = pltpu.PrefetchScalarGridSpec(
        num_scalar_prefetch=1,
        grid=(NSTEP,),
        in_specs=in_specs,
        out_specs=pl.BlockSpec((KT * TILE, DIN), lambda i, te: (i, 0)),
    )
    return pl.pallas_call(
        _mlp_body,
        grid_spec=grid_spec,
        out_shape=jax.ShapeDtypeStruct((NPAD, DIN), jnp.float32),
    )(texp, xpad, *operands)


def kernel(input, W1, b1, W2, b2, W3, b3, W4, b4, W5, b5, W6, b6, W7, b7):
    inp2d = input.reshape(NTOK, DIN + ID_LEN)
    sc_scatter, sc_gather = _sc_kernels()
    pos2d, texp = _routing(input.reshape(NBLK, TILE, DIN + ID_LEN))
    pos = pos2d.reshape(NTOK)
    x = inp2d[:, :DIN]
    xpad = sc_scatter(x, pos)
    ypad = _mlp(texp, xpad, W1, b1, W2, b2, W3, b3, W4, b4, W5, b5,
                W6, b6, W7, b7)
    out = sc_gather(ypad, pos)
    return out[:, :ACTS].reshape(NB, NA, ACTS)


# speedup vs baseline: 9.9695x; 1.8176x over previous
import functools

import jax
import jax.numpy as jnp
from jax import lax
from jax.experimental import pallas as pl
from jax.experimental.pallas import tpu as pltpu
from jax.experimental.pallas import tpu_sc as plsc

ID_LEN = 64
DIN = 128
NB = 4
NA = 512
NTOK = NB * NA
NE = 64
TILE = 128
NBLK = NTOK // TILE
NT = NBLK + NE
NPAD = NT * TILE
ACTS = 64
NW = 32
RPW = NTOK // NW


def _routing_body(inp_ref, pos_ref, texp_ref):
    row_i = lax.broadcasted_iota(jnp.int32, (TILE, TILE), 0)
    col_i = lax.broadcasted_iota(jnp.int32, (TILE, TILE), 1)
    lstrict = (row_i > col_i).astype(jnp.float32)
    iota_e = lax.broadcasted_iota(jnp.int32, (TILE, ID_LEN), 1)

    onehots = []
    rels = []
    csums = []
    for b in range(NBLK):
        tail = inp_ref[b][:, DIN:]
        m = jnp.max(tail, axis=1, keepdims=True)
        ids_b = jnp.min(jnp.where(tail == m, iota_e, ID_LEN), axis=1)
        onehot = (iota_e == ids_b[:, None]).astype(jnp.float32)
        rel = jnp.sum(jnp.dot(lstrict, onehot,
                              preferred_element_type=jnp.float32) * onehot, axis=1)
        onehots.append(onehot)
        rels.append(rel)
        csums.append(jnp.sum(onehot, axis=0, keepdims=True))

    C = jnp.concatenate(csums, axis=0)
    b_row = lax.broadcasted_iota(jnp.int32, (NBLK, NBLK), 0)
    b_col = lax.broadcasted_iota(jnp.int32, (NBLK, NBLK), 1)
    lb = (b_row > b_col).astype(jnp.float32)
    P = jnp.dot(lb, C, preferred_element_type=jnp.float32)
    ranks = [rels[b] + jnp.sum(onehots[b] * P[b:b + 1, :], axis=1)
             for b in range(NBLK)]

    cnt = jnp.sum(C, axis=0)
    tiles_e = jnp.floor((cnt + (TILE - 1)) * (1.0 / TILE))
    e_row = lax.broadcasted_iota(jnp.int32, (NE, NE), 0)
    e_col = lax.broadcasted_iota(jnp.int32, (NE, NE), 1)
    l64 = (e_row > e_col).astype(jnp.float32)
    start_t = jnp.dot(l64, tiles_e, preferred_element_type=jnp.float32)
    offsets = start_t * TILE
    for b in range(NBLK):
        pos_b = jnp.sum(onehots[b] * offsets[None, :], axis=1) + ranks[b]
        pos_ref[pl.ds(b, 1), :] = pos_b.astype(jnp.int32)[None, :]
    t_row = lax.broadcasted_iota(jnp.int32, (NT, NE), 0).astype(jnp.float32)
    le = (start_t[None, :] <= t_row).astype(jnp.float32)
    texp_ref[...] = (jnp.sum(le, axis=1) - 1.0).astype(jnp.int32)


def _routing(inp3d):
    return pl.pallas_call(
        _routing_body,
        out_shape=[jax.ShapeDtypeStruct((NBLK, TILE), jnp.int32),
                   jax.ShapeDtypeStruct((NT,), jnp.int32)],
    )(inp3d)


@functools.lru_cache(maxsize=None)
def _sc_kernels():
    mesh = plsc.VectorSubcoreMesh(core_axis_name="c", subcore_axis_name="s")

    @functools.partial(
        pl.kernel,
        out_type=jax.ShapeDtypeStruct((NPAD, DIN), jnp.float32),
        mesh=mesh,
        scratch_types=[pltpu.VMEM((RPW,), jnp.int32),
                       pltpu.VMEM((RPW, DIN), jnp.float32),
                       pltpu.SemaphoreType.DMA],
    )
    def sc_scatter(x_hbm, pos_hbm, xpad_hbm, idx_v, rows_v, sem):
        wid = lax.axis_index("s") * mesh.num_cores + lax.axis_index("c")
        base = wid * RPW
        pltpu.sync_copy(x_hbm.at[pl.ds(base, RPW)], rows_v)
        pltpu.sync_copy(pos_hbm.at[pl.ds(base, RPW)], idx_v)
        pltpu.async_copy(rows_v, xpad_hbm.at[idx_v], sem).wait()

    @functools.partial(
        pl.kernel,
        out_type=jax.ShapeDtypeStruct((NTOK, DIN), jnp.float32),
        mesh=mesh,
        scratch_types=[pltpu.VMEM((RPW,), jnp.int32),
                       pltpu.VMEM((RPW, DIN), jnp.float32),
                       pltpu.SemaphoreType.DMA],
    )
    def sc_gather(ypad_hbm, pos_hbm, out_hbm, idx_v, rows_v, sem):
        wid = lax.axis_index("s") * mesh.num_cores + lax.axis_index("c")
        base = wid * RPW
        pltpu.sync_copy(pos_hbm.at[pl.ds(base, RPW)], idx_v)
        pltpu.async_copy(ypad_hbm.at[idx_v], rows_v, sem).wait()
        pltpu.sync_copy(rows_v, out_hbm.at[pl.ds(base, RPW)])

    return sc_scatter, sc_gather


KT = 8
NSTEP = NT // KT


def _mlp_body(te_ref, x_ref, w1, b1, w2, b2, w3, b3, w4, b4, w5, b5,
              w6, b6, w7, b7, y_ref):
    i = pl.program_id(0)
    tes = [te_ref[KT * i + j] for j in range(KT)]

    def lay(x, wr, br, te_j, relu):
        w = wr[te_j]
        y = lax.dot_general(x, w, (((1,), (1,)), ((), ())),
                            preferred_element_type=jnp.float32) + br[te_j]
        return jnp.maximum(y, 0.0) if relu else y

    hs = [x_ref[pl.ds(j * TILE, TILE), :] for j in range(KT)]
    for (wr, br, relu) in ((w1, b1, True), (w2, b2, True), (w3, b3, False),
                           (w4, b4, True), (w5, b5, True), (w6, b6, False),
                           (w7, b7, False)):
        hs = [lay(hs[j], wr, br, tes[j], relu) for j in range(KT)]
    for j in range(KT):
        y_ref[pl.ds(j * TILE, TILE), :] = jnp.concatenate(
            [hs[j], jnp.zeros((TILE, DIN - ACTS), jnp.float32)], axis=1)


def _mlp(texp, xpad, W1, b1, W2, b2, W3, b3, W4, b4, W5, b5, W6, b6, W7, b7):
    def wspec(out_dim):
        return pl.BlockSpec((NE, out_dim, DIN), lambda i, te: (0, 0, 0))

    def bspec(out_dim):
        return pl.BlockSpec((NE, 1, out_dim), lambda i, te: (0, 0, 0))

    in_specs = [pl.BlockSpec((KT * TILE, DIN), lambda i, te: (i, 0)),
                wspec(128), bspec(128), wspec(128), bspec(128),
                wspec(128), bspec(128), wspec(128), bspec(128),
                wspec(128), bspec(128), wspec(128), bspec(128),
                wspec(ACTS), bspec(ACTS)]

    grid_spec = pltpu.PrefetchScalarGridSpec(
        num_scalar_prefetch=1,
        grid=(NSTEP,),
        in_specs=in_specs,
        out_specs=pl.BlockSpec((KT * TILE, DIN), lambda i, te: (i, 0)),
    )
    return pl.pallas_call(
        _mlp_body,
        grid_spec=grid_spec,
        out_shape=jax.ShapeDtypeStruct((NPAD, DIN), jnp.float32),
    )(texp, xpad,
      W1, b1.reshape(NE, 1, -1), W2, b2.reshape(NE, 1, -1),
      W3, b3.reshape(NE, 1, -1), W4, b4.reshape(NE, 1, -1),
      W5, b5.reshape(NE, 1, -1), W6, b6.reshape(NE, 1, -1),
      W7, b7.reshape(NE, 1, -1))


def kernel(input, W1, b1, W2, b2, W3, b3, W4, b4, W5, b5, W6, b6, W7, b7):
    inp2d = input.reshape(NTOK, DIN + ID_LEN)
    sc_scatter, sc_gather = _sc_kernels()
    pos2d, texp = _routing(input.reshape(NBLK, TILE, DIN + ID_LEN))
    pos = pos2d.reshape(NTOK)
    x = inp2d[:, :DIN]
    xpad = sc_scatter(x, pos)
    ypad = _mlp(texp, xpad, W1, b1, W2, b2, W3, b3, W4, b4, W5, b5,
                W6, b6, W7, b7)
    out = sc_gather(ypad, pos)
    return out[:, :ACTS].reshape(NB, NA, ACTS)


# speedup vs baseline: 10.3001x; 1.0332x over previous
import functools

import jax
import jax.numpy as jnp
from jax import lax
from jax.experimental import pallas as pl
from jax.experimental.pallas import tpu as pltpu
from jax.experimental.pallas import tpu_sc as plsc

ID_LEN = 64
DIN = 128
NB = 4
NA = 512
NTOK = NB * NA
NE = 64
TILE = 128
NBLK = NTOK // TILE
NT = NBLK + NE
NPAD = NT * TILE
ACTS = 64
NW = 32
RPW = NTOK // NW


def _routing_body(inp_ref, pos_ref, texp_ref, x_ref):
    row_i = lax.broadcasted_iota(jnp.int32, (TILE, TILE), 0)
    col_i = lax.broadcasted_iota(jnp.int32, (TILE, TILE), 1)
    lstrict = (row_i > col_i).astype(jnp.float32)
    iota_e = lax.broadcasted_iota(jnp.int32, (TILE, ID_LEN), 1)

    onehots = []
    rels = []
    csums = []
    for b in range(NBLK):
        blk = inp_ref[b]
        x_ref[b] = blk[:, :DIN]
        tail = blk[:, DIN:]
        m = jnp.max(tail, axis=1, keepdims=True)
        ids_b = jnp.min(jnp.where(tail == m, iota_e, ID_LEN), axis=1)
        onehot = (iota_e == ids_b[:, None]).astype(jnp.float32)
        rel = jnp.sum(jnp.dot(lstrict, onehot,
                              preferred_element_type=jnp.float32) * onehot, axis=1)
        onehots.append(onehot)
        rels.append(rel)
        csums.append(jnp.sum(onehot, axis=0, keepdims=True))

    C = jnp.concatenate(csums, axis=0)
    b_row = lax.broadcasted_iota(jnp.int32, (NBLK, NBLK), 0)
    b_col = lax.broadcasted_iota(jnp.int32, (NBLK, NBLK), 1)
    lb = (b_row > b_col).astype(jnp.float32)
    P = jnp.dot(lb, C, preferred_element_type=jnp.float32)
    ranks = [rels[b] + jnp.sum(onehots[b] * P[b:b + 1, :], axis=1)
             for b in range(NBLK)]

    cnt = jnp.sum(C, axis=0)
    tiles_e = jnp.floor((cnt + (TILE - 1)) * (1.0 / TILE))
    e_row = lax.broadcasted_iota(jnp.int32, (NE, NE), 0)
    e_col = lax.broadcasted_iota(jnp.int32, (NE, NE), 1)
    l64 = (e_row > e_col).astype(jnp.float32)
    start_t = jnp.dot(l64, tiles_e, preferred_element_type=jnp.float32)
    offsets = start_t * TILE
    for b in range(NBLK):
        pos_b = jnp.sum(onehots[b] * offsets[None, :], axis=1) + ranks[b]
        pos_ref[pl.ds(b, 1), :] = pos_b.astype(jnp.int32)[None, :]
    t_row = lax.broadcasted_iota(jnp.int32, (NT, NE), 0).astype(jnp.float32)
    le = (start_t[None, :] <= t_row).astype(jnp.float32)
    texp_ref[...] = (jnp.sum(le, axis=1) - 1.0).astype(jnp.int32)


def _routing(inp3d):
    return pl.pallas_call(
        _routing_body,
        out_shape=[jax.ShapeDtypeStruct((NBLK, TILE), jnp.int32),
                   jax.ShapeDtypeStruct((NT,), jnp.int32),
                   jax.ShapeDtypeStruct((NBLK, TILE, DIN), jnp.float32)],
    )(inp3d)


@functools.lru_cache(maxsize=None)
def _sc_kernels():
    mesh = plsc.VectorSubcoreMesh(core_axis_name="c", subcore_axis_name="s")

    @functools.partial(
        pl.kernel,
        out_type=jax.ShapeDtypeStruct((NPAD, DIN), jnp.float32),
        mesh=mesh,
        scratch_types=[pltpu.VMEM((RPW,), jnp.int32),
                       pltpu.VMEM((RPW, DIN), jnp.float32),
                       pltpu.SemaphoreType.DMA],
    )
    def sc_scatter(x_hbm, pos_hbm, xpad_hbm, idx_v, rows_v, sem):
        wid = lax.axis_index("s") * mesh.num_cores + lax.axis_index("c")
        base = wid * RPW
        pltpu.sync_copy(x_hbm.at[pl.ds(base, RPW)], rows_v)
        pltpu.sync_copy(pos_hbm.at[pl.ds(base, RPW)], idx_v)
        pltpu.async_copy(rows_v, xpad_hbm.at[idx_v], sem).wait()

    @functools.partial(
        pl.kernel,
        out_type=jax.ShapeDtypeStruct((NTOK, DIN), jnp.float32),
        mesh=mesh,
        scratch_types=[pltpu.VMEM((RPW,), jnp.int32),
                       pltpu.VMEM((RPW, DIN), jnp.float32),
                       pltpu.SemaphoreType.DMA],
    )
    def sc_gather(ypad_hbm, pos_hbm, out_hbm, idx_v, rows_v, sem):
        wid = lax.axis_index("s") * mesh.num_cores + lax.axis_index("c")
        base = wid * RPW
        pltpu.sync_copy(pos_hbm.at[pl.ds(base, RPW)], idx_v)
        pltpu.async_copy(ypad_hbm.at[idx_v], rows_v, sem).wait()
        pltpu.sync_copy(rows_v, out_hbm.at[pl.ds(base, RPW)])

    return sc_scatter, sc_gather


KT = 8
NSTEP = NT // KT


def _mlp_body(te_ref, x_ref, w1, b1, w2, b2, w3, b3, w4, b4, w5, b5,
              w6, b6, w7, b7, y_ref):
    i = pl.program_id(0)
    tes = [te_ref[KT * i + j] for j in range(KT)]

    def lay(x, wr, br, te_j, relu):
        w = wr[te_j]
        y = lax.dot_general(x, w, (((1,), (1,)), ((), ())),
                            preferred_element_type=jnp.float32) + br[te_j]
        return jnp.maximum(y, 0.0) if relu else y

    hs = [x_ref[pl.ds(j * TILE, TILE), :] for j in range(KT)]
    for (wr, br, relu) in ((w1, b1, True), (w2, b2, True), (w3, b3, False),
                           (w4, b4, True), (w5, b5, True), (w6, b6, False),
                           (w7, b7, False)):
        hs = [lay(hs[j], wr, br, tes[j], relu) for j in range(KT)]
    for j in range(KT):
        y_ref[pl.ds(j * TILE, TILE), :] = jnp.concatenate(
            [hs[j], jnp.zeros((TILE, DIN - ACTS), jnp.float32)], axis=1)


def _mlp(texp, xpad, W1, b1, W2, b2, W3, b3, W4, b4, W5, b5, W6, b6, W7, b7):
    def wspec(out_dim):
        return pl.BlockSpec((NE, out_dim, DIN), lambda i, te: (0, 0, 0))

    def bspec(out_dim):
        return pl.BlockSpec((NE, 1, out_dim), lambda i, te: (0, 0, 0))

    in_specs = [pl.BlockSpec((KT * TILE, DIN), lambda i, te: (i, 0)),
                wspec(128), bspec(128), wspec(128), bspec(128),
                wspec(128), bspec(128), wspec(128), bspec(128),
                wspec(128), bspec(128), wspec(128), bspec(128),
                wspec(ACTS), bspec(ACTS)]

    grid_spec = pltpu.PrefetchScalarGridSpec(
        num_scalar_prefetch=1,
        grid=(NSTEP,),
        in_specs=in_specs,
        out_specs=pl.BlockSpec((KT * TILE, DIN), lambda i, te: (i, 0)),
    )
    return pl.pallas_call(
        _mlp_body,
        grid_spec=grid_spec,
        out_shape=jax.ShapeDtypeStruct((NPAD, DIN), jnp.float32),
    )(texp, xpad,
      W1, b1.reshape(NE, 1, -1), W2, b2.reshape(NE, 1, -1),
      W3, b3.reshape(NE, 1, -1), W4, b4.reshape(NE, 1, -1),
      W5, b5.reshape(NE, 1, -1), W6, b6.reshape(NE, 1, -1),
      W7, b7.reshape(NE, 1, -1))


def kernel(input, W1, b1, W2, b2, W3, b3, W4, b4, W5, b5, W6, b6, W7, b7):
    sc_scatter, sc_gather = _sc_kernels()
    pos2d, texp, x3d = _routing(input.reshape(NBLK, TILE, DIN + ID_LEN))
    pos = pos2d.reshape(NTOK)
    x = x3d.reshape(NTOK, DIN)
    xpad = sc_scatter(x, pos)
    ypad = _mlp(texp, xpad, W1, b1, W2, b2, W3, b3, W4, b4, W5, b5,
                W6, b6, W7, b7)
    out = sc_gather(ypad, pos)
    return out[:, :ACTS].reshape(NB, NA, ACTS)


# speedup vs baseline: 10.3960x; 1.0093x over previous
import functools

import jax
import jax.numpy as jnp
from jax import lax
from jax.experimental import pallas as pl
from jax.experimental.pallas import tpu as pltpu
from jax.experimental.pallas import tpu_sc as plsc

ID_LEN = 64
DIN = 128
NB = 4
NA = 512
NTOK = NB * NA
NE = 64
TILE = 128
NBLK = NTOK // TILE
NT = NBLK + NE
NPAD = NT * TILE
ACTS = 64
NW = 32
RPW = NTOK // NW


def _routing_body(inp_ref, pos_ref, texp_ref, x_ref):
    row_i = lax.broadcasted_iota(jnp.int32, (TILE, TILE), 0)
    col_i = lax.broadcasted_iota(jnp.int32, (TILE, TILE), 1)
    lstrict = (row_i > col_i).astype(jnp.float32)
    iota_e = lax.broadcasted_iota(jnp.int32, (TILE, ID_LEN), 1)

    onehots = []
    rels = []
    csums = []
    for b in range(NBLK):
        blk = inp_ref[b]
        x_ref[b] = blk[:, :DIN]
        tail = blk[:, DIN:]
        m = jnp.max(tail, axis=1, keepdims=True)
        ids_b = jnp.min(jnp.where(tail == m, iota_e, ID_LEN), axis=1)
        onehot = (iota_e == ids_b[:, None]).astype(jnp.float32)
        rel = jnp.sum(jnp.dot(lstrict, onehot,
                              preferred_element_type=jnp.float32) * onehot, axis=1)
        onehots.append(onehot)
        rels.append(rel)
        csums.append(jnp.sum(onehot, axis=0, keepdims=True))

    C = jnp.concatenate(csums, axis=0)
    b_row = lax.broadcasted_iota(jnp.int32, (NBLK, NBLK), 0)
    b_col = lax.broadcasted_iota(jnp.int32, (NBLK, NBLK), 1)
    lb = (b_row > b_col).astype(jnp.float32)
    P = jnp.dot(lb, C, preferred_element_type=jnp.float32)
    ranks = [rels[b] + jnp.sum(onehots[b] * P[b:b + 1, :], axis=1)
             for b in range(NBLK)]

    cnt = jnp.sum(C, axis=0)
    tiles_e = jnp.floor((cnt + (TILE - 1)) * (1.0 / TILE))
    e_row = lax.broadcasted_iota(jnp.int32, (NE, NE), 0)
    e_col = lax.broadcasted_iota(jnp.int32, (NE, NE), 1)
    l64 = (e_row > e_col).astype(jnp.float32)
    start_t = jnp.dot(l64, tiles_e, preferred_element_type=jnp.float32)
    offsets = start_t * TILE
    for b in range(NBLK):
        pos_b = jnp.sum(onehots[b] * offsets[None, :], axis=1) + ranks[b]
        pos_ref[pl.ds(b, 1), :] = pos_b.astype(jnp.int32)[None, :]
    t_row = lax.broadcasted_iota(jnp.int32, (NT, NE), 0).astype(jnp.float32)
    le = (start_t[None, :] <= t_row).astype(jnp.float32)
    texp_ref[...] = (jnp.sum(le, axis=1) - 1.0).astype(jnp.int32)


def _routing(inp3d):
    return pl.pallas_call(
        _routing_body,
        out_shape=[jax.ShapeDtypeStruct((NBLK, TILE), jnp.int32),
                   jax.ShapeDtypeStruct((NT,), jnp.int32),
                   jax.ShapeDtypeStruct((NBLK, TILE, DIN), jnp.float32)],
    )(inp3d)


@functools.lru_cache(maxsize=None)
def _sc_kernels():
    mesh = plsc.VectorSubcoreMesh(core_axis_name="c", subcore_axis_name="s")

    @functools.partial(
        pl.kernel,
        out_type=jax.ShapeDtypeStruct((NPAD, DIN), jnp.float32),
        mesh=mesh,
        scratch_types=[pltpu.VMEM((RPW,), jnp.int32),
                       pltpu.VMEM((RPW, DIN), jnp.float32),
                       pltpu.SemaphoreType.DMA],
    )
    def sc_scatter(x_hbm, pos_hbm, xpad_hbm, idx_v, rows_v, sem):
        wid = lax.axis_index("s") * mesh.num_cores + lax.axis_index("c")
        base = wid * RPW
        pltpu.sync_copy(x_hbm.at[pl.ds(base, RPW)], rows_v)
        pltpu.sync_copy(pos_hbm.at[pl.ds(base, RPW)], idx_v)
        pltpu.async_copy(rows_v, xpad_hbm.at[idx_v], sem).wait()

    @functools.partial(
        pl.kernel,
        out_type=jax.ShapeDtypeStruct((NTOK, DIN), jnp.float32),
        mesh=mesh,
        scratch_types=[pltpu.VMEM((RPW,), jnp.int32),
                       pltpu.VMEM((RPW, DIN), jnp.float32),
                       pltpu.SemaphoreType.DMA],
    )
    def sc_gather(ypad_hbm, pos_hbm, out_hbm, idx_v, rows_v, sem):
        wid = lax.axis_index("s") * mesh.num_cores + lax.axis_index("c")
        base = wid * RPW
        pltpu.sync_copy(pos_hbm.at[pl.ds(base, RPW)], idx_v)
        pltpu.async_copy(ypad_hbm.at[idx_v], rows_v, sem).wait()
        pltpu.sync_copy(rows_v, out_hbm.at[pl.ds(base, RPW)])

    return sc_scatter, sc_gather


KT = 16
NSTEP = NT // KT


def _mlp_body(te_ref, x_ref, w1, b1, w2, b2, w3, b3, w4, b4, w5, b5,
              w6, b6, w7, b7, y_ref):
    i = pl.program_id(0)
    tes = [te_ref[KT * i + j] for j in range(KT)]

    def lay(x, wr, br, te_j, relu):
        w = wr[te_j]
        y = lax.dot_general(x, w, (((1,), (1,)), ((), ())),
                            preferred_element_type=jnp.float32) + br[te_j]
        return jnp.maximum(y, 0.0) if relu else y

    hs = [x_ref[pl.ds(j * TILE, TILE), :] for j in range(KT)]
    for (wr, br, relu) in ((w1, b1, True), (w2, b2, True), (w3, b3, False),
                           (w4, b4, True), (w5, b5, True), (w6, b6, False),
                           (w7, b7, False)):
        hs = [lay(hs[j], wr, br, tes[j], relu) for j in range(KT)]
    for j in range(KT):
        y_ref[pl.ds(j * TILE, TILE), :] = jnp.concatenate(
            [hs[j], jnp.zeros((TILE, DIN - ACTS), jnp.float32)], axis=1)


def _mlp(texp, xpad, W1, b1, W2, b2, W3, b3, W4, b4, W5, b5, W6, b6, W7, b7):
    def wspec(out_dim):
        return pl.BlockSpec((NE, out_dim, DIN), lambda i, te: (0, 0, 0))

    def bspec(out_dim):
        return pl.BlockSpec((NE, 1, out_dim), lambda i, te: (0, 0, 0))

    in_specs = [pl.BlockSpec((KT * TILE, DIN), lambda i, te: (i, 0)),
                wspec(128), bspec(128), wspec(128), bspec(128),
                wspec(128), bspec(128), wspec(128), bspec(128),
                wspec(128), bspec(128), wspec(128), bspec(128),
                wspec(ACTS), bspec(ACTS)]

    grid_spec = pltpu.PrefetchScalarGridSpec(
        num_scalar_prefetch=1,
        grid=(NSTEP,),
        in_specs=in_specs,
        out_specs=pl.BlockSpec((KT * TILE, DIN), lambda i, te: (i, 0)),
    )
    return pl.pallas_call(
        _mlp_body,
        grid_spec=grid_spec,
        out_shape=jax.ShapeDtypeStruct((NPAD, DIN), jnp.float32),
    )(texp, xpad,
      W1, b1.reshape(NE, 1, -1), W2, b2.reshape(NE, 1, -1),
      W3, b3.reshape(NE, 1, -1), W4, b4.reshape(NE, 1, -1),
      W5, b5.reshape(NE, 1, -1), W6, b6.reshape(NE, 1, -1),
      W7, b7.reshape(NE, 1, -1))


def kernel(input, W1, b1, W2, b2, W3, b3, W4, b4, W5, b5, W6, b6, W7, b7):
    sc_scatter, sc_gather = _sc_kernels()
    pos2d, texp, x3d = _routing(input.reshape(NBLK, TILE, DIN + ID_LEN))
    pos = pos2d.reshape(NTOK)
    x = x3d.reshape(NTOK, DIN)
    xpad = sc_scatter(x, pos)
    ypad = _mlp(texp, xpad, W1, b1, W2, b2, W3, b3, W4, b4, W5, b5,
                W6, b6, W7, b7)
    out = sc_gather(ypad, pos)
    return out[:, :ACTS].reshape(NB, NA, ACTS)


# speedup vs baseline: 10.4426x; 1.0045x over previous
import functools

import jax
import jax.numpy as jnp
from jax import lax
from jax.experimental import pallas as pl
from jax.experimental.pallas import tpu as pltpu
from jax.experimental.pallas import tpu_sc as plsc

ID_LEN = 64
DIN = 128
NB = 4
NA = 512
NTOK = NB * NA
NE = 64
TILE = 128
NBLK = NTOK // TILE
NT = NBLK + NE
NPAD = NT * TILE
ACTS = 64
NW = 32
RPW = NTOK // NW


def _routing_body(inp_ref, pos_ref, texp_ref, x_ref):
    row_i = lax.broadcasted_iota(jnp.int32, (TILE, TILE), 0)
    col_i = lax.broadcasted_iota(jnp.int32, (TILE, TILE), 1)
    lstrict = (row_i > col_i).astype(jnp.float32)
    iota_e = lax.broadcasted_iota(jnp.int32, (TILE, ID_LEN), 1)

    onehots = []
    rels = []
    csums = []
    for b in range(NBLK):
        blk = inp_ref[b]
        x_ref[b] = blk[:, :DIN]
        tail = blk[:, DIN:]
        m = jnp.max(tail, axis=1, keepdims=True)
        ids_b = jnp.min(jnp.where(tail == m, iota_e, ID_LEN), axis=1)
        onehot = (iota_e == ids_b[:, None]).astype(jnp.float32)
        rel = jnp.sum(jnp.dot(lstrict, onehot,
                              preferred_element_type=jnp.float32) * onehot, axis=1)
        onehots.append(onehot)
        rels.append(rel)
        csums.append(jnp.sum(onehot, axis=0, keepdims=True))

    C = jnp.concatenate(csums, axis=0)
    b_row = lax.broadcasted_iota(jnp.int32, (NBLK, NBLK), 0)
    b_col = lax.broadcasted_iota(jnp.int32, (NBLK, NBLK), 1)
    lb = (b_row > b_col).astype(jnp.float32)
    P = jnp.dot(lb, C, preferred_element_type=jnp.float32)
    ranks = [rels[b] + jnp.sum(onehots[b] * P[b:b + 1, :], axis=1)
             for b in range(NBLK)]

    cnt = jnp.sum(C, axis=0)
    tiles_e = jnp.floor((cnt + (TILE - 1)) * (1.0 / TILE))
    e_row = lax.broadcasted_iota(jnp.int32, (NE, NE), 0)
    e_col = lax.broadcasted_iota(jnp.int32, (NE, NE), 1)
    l64 = (e_row > e_col).astype(jnp.float32)
    start_t = jnp.dot(l64, tiles_e, preferred_element_type=jnp.float32)
    offsets = start_t * TILE
    for b in range(NBLK):
        pos_b = jnp.sum(onehots[b] * offsets[None, :], axis=1) + ranks[b]
        pos_ref[pl.ds(b, 1), :] = pos_b.astype(jnp.int32)[None, :]
    t_row = lax.broadcasted_iota(jnp.int32, (NT, NE), 0).astype(jnp.float32)
    le = (start_t[None, :] <= t_row).astype(jnp.float32)
    texp_ref[...] = (jnp.sum(le, axis=1) - 1.0).astype(jnp.int32)


def _routing(inp3d):
    return pl.pallas_call(
        _routing_body,
        out_shape=[jax.ShapeDtypeStruct((NBLK, TILE), jnp.int32),
                   jax.ShapeDtypeStruct((NT,), jnp.int32),
                   jax.ShapeDtypeStruct((NBLK, TILE, DIN), jnp.float32)],
    )(inp3d)


@functools.lru_cache(maxsize=None)
def _sc_kernels():
    mesh = plsc.VectorSubcoreMesh(core_axis_name="c", subcore_axis_name="s")

    @functools.partial(
        pl.kernel,
        out_type=jax.ShapeDtypeStruct((NPAD, DIN), jnp.float32),
        mesh=mesh,
        scratch_types=[pltpu.VMEM((RPW,), jnp.int32),
                       pltpu.VMEM((RPW, DIN), jnp.float32),
                       pltpu.SemaphoreType.DMA],
    )
    def sc_scatter(x_hbm, pos_hbm, xpad_hbm, idx_v, rows_v, sem):
        wid = lax.axis_index("s") * mesh.num_cores + lax.axis_index("c")
        base = wid * RPW
        pltpu.sync_copy(x_hbm.at[pl.ds(base, RPW)], rows_v)
        pltpu.sync_copy(pos_hbm.at[pl.ds(base, RPW)], idx_v)
        pltpu.async_copy(rows_v, xpad_hbm.at[idx_v], sem).wait()

    @functools.partial(
        pl.kernel,
        out_type=jax.ShapeDtypeStruct((NTOK, DIN), jnp.float32),
        mesh=mesh,
        scratch_types=[pltpu.VMEM((RPW,), jnp.int32),
                       pltpu.VMEM((RPW, DIN), jnp.float32),
                       pltpu.SemaphoreType.DMA],
    )
    def sc_gather(ypad_hbm, pos_hbm, out_hbm, idx_v, rows_v, sem):
        wid = lax.axis_index("s") * mesh.num_cores + lax.axis_index("c")
        base = wid * RPW
        pltpu.sync_copy(pos_hbm.at[pl.ds(base, RPW)], idx_v)
        pltpu.async_copy(ypad_hbm.at[idx_v], rows_v, sem).wait()
        pltpu.sync_copy(rows_v, out_hbm.at[pl.ds(base, RPW)])

    return sc_scatter, sc_gather


KT = 20
NSTEP = NT // KT


def _mlp_body(te_ref, x_ref, w1, b1, w2, b2, w3, b3, w4, b4, w5, b5,
              w6, b6, w7, b7, y_ref):
    i = pl.program_id(0)
    tes = [te_ref[KT * i + j] for j in range(KT)]

    def lay(x, wr, br, te_j, relu):
        w = wr[te_j]
        y = lax.dot_general(x, w, (((1,), (1,)), ((), ())),
                            preferred_element_type=jnp.float32) + br[te_j]
        return jnp.maximum(y, 0.0) if relu else y

    hs = [x_ref[pl.ds(j * TILE, TILE), :] for j in range(KT)]
    for (wr, br, relu) in ((w1, b1, True), (w2, b2, True), (w3, b3, False),
                           (w4, b4, True), (w5, b5, True), (w6, b6, False),
                           (w7, b7, False)):
        hs = [lay(hs[j], wr, br, tes[j], relu) for j in range(KT)]
    for j in range(KT):
        y_ref[pl.ds(j * TILE, TILE), :] = jnp.concatenate(
            [hs[j], jnp.zeros((TILE, DIN - ACTS), jnp.float32)], axis=1)


def _mlp(texp, xpad, W1, b1, W2, b2, W3, b3, W4, b4, W5, b5, W6, b6, W7, b7):
    def wspec(out_dim):
        return pl.BlockSpec((NE, out_dim, DIN), lambda i, te: (0, 0, 0))

    def bspec(out_dim):
        return pl.BlockSpec((NE, 1, out_dim), lambda i, te: (0, 0, 0))

    in_specs = [pl.BlockSpec((KT * TILE, DIN), lambda i, te: (i, 0)),
                wspec(128), bspec(128), wspec(128), bspec(128),
                wspec(128), bspec(128), wspec(128), bspec(128),
                wspec(128), bspec(128), wspec(128), bspec(128),
                wspec(ACTS), bspec(ACTS)]

    grid_spec = pltpu.PrefetchScalarGridSpec(
        num_scalar_prefetch=1,
        grid=(NSTEP,),
        in_specs=in_specs,
        out_specs=pl.BlockSpec((KT * TILE, DIN), lambda i, te: (i, 0)),
    )
    return pl.pallas_call(
        _mlp_body,
        grid_spec=grid_spec,
        out_shape=jax.ShapeDtypeStruct((NPAD, DIN), jnp.float32),
    )(texp, xpad,
      W1, b1.reshape(NE, 1, -1), W2, b2.reshape(NE, 1, -1),
      W3, b3.reshape(NE, 1, -1), W4, b4.reshape(NE, 1, -1),
      W5, b5.reshape(NE, 1, -1), W6, b6.reshape(NE, 1, -1),
      W7, b7.reshape(NE, 1, -1))


def kernel(input, W1, b1, W2, b2, W3, b3, W4, b4, W5, b5, W6, b6, W7, b7):
    sc_scatter, sc_gather = _sc_kernels()
    pos2d, texp, x3d = _routing(input.reshape(NBLK, TILE, DIN + ID_LEN))
    pos = pos2d.reshape(NTOK)
    x = x3d.reshape(NTOK, DIN)
    xpad = sc_scatter(x, pos)
    ypad = _mlp(texp, xpad, W1, b1, W2, b2, W3, b3, W4, b4, W5, b5,
                W6, b6, W7, b7)
    out = sc_gather(ypad, pos)
    return out[:, :ACTS].reshape(NB, NA, ACTS)


# speedup vs baseline: 10.7053x; 1.0251x over previous
import functools

import jax
import jax.numpy as jnp
from jax import lax
from jax.experimental import pallas as pl
from jax.experimental.pallas import tpu as pltpu
from jax.experimental.pallas import tpu_sc as plsc

ID_LEN = 64
DIN = 128
NB = 4
NA = 512
NTOK = NB * NA
NE = 64
TILE = 128
NBLK = NTOK // TILE
NT = NBLK + NE
NPAD = NT * TILE
ACTS = 64
NW = 16
RPW = NTOK // NW


def _routing_body(inp_ref, pos_ref, texp_ref, x_ref):
    row_i = lax.broadcasted_iota(jnp.int32, (TILE, TILE), 0)
    col_i = lax.broadcasted_iota(jnp.int32, (TILE, TILE), 1)
    lstrict = (row_i > col_i).astype(jnp.float32)
    iota_e = lax.broadcasted_iota(jnp.int32, (TILE, ID_LEN), 1)

    onehots = []
    csums = []
    for b in range(NBLK):
        blk = inp_ref[b]
        x_ref[b] = blk[:, :DIN]
        tail = blk[:, DIN:]
        m = jnp.max(tail, axis=1, keepdims=True)
        ids_b = jnp.min(jnp.where(tail == m, iota_e, ID_LEN), axis=1)
        onehot = (iota_e == ids_b[:, None]).astype(jnp.float32)
        onehots.append(onehot)
        csums.append(jnp.sum(onehot, axis=0, keepdims=True))

    O_all = jnp.concatenate(onehots, axis=1)
    R_all = jnp.dot(lstrict, O_all, preferred_element_type=jnp.float32)
    rels = [jnp.sum(R_all[:, b * NE:(b + 1) * NE] * onehots[b], axis=1)
            for b in range(NBLK)]

    C = jnp.concatenate(csums, axis=0)
    b_row = lax.broadcasted_iota(jnp.int32, (NBLK, NBLK), 0)
    b_col = lax.broadcasted_iota(jnp.int32, (NBLK, NBLK), 1)
    lb = (b_row > b_col).astype(jnp.float32)
    P = jnp.dot(lb, C, preferred_element_type=jnp.float32)
    ranks = [rels[b] + jnp.sum(onehots[b] * P[b:b + 1, :], axis=1)
             for b in range(NBLK)]

    cnt = jnp.sum(C, axis=0)
    tiles_e = jnp.floor((cnt + (TILE - 1)) * (1.0 / TILE))
    e_row = lax.broadcasted_iota(jnp.int32, (NE, NE), 0)
    e_col = lax.broadcasted_iota(jnp.int32, (NE, NE), 1)
    l64 = (e_row > e_col).astype(jnp.float32)
    start_t = jnp.dot(l64, tiles_e, preferred_element_type=jnp.float32)
    offsets = start_t * TILE
    for b in range(NBLK):
        pos_b = jnp.sum(onehots[b] * offsets[None, :], axis=1) + ranks[b]
        pos_ref[pl.ds(b, 1), :] = pos_b.astype(jnp.int32)[None, :]
    t_row = lax.broadcasted_iota(jnp.int32, (NT, NE), 0).astype(jnp.float32)
    le = (start_t[None, :] <= t_row).astype(jnp.float32)
    texp_ref[...] = (jnp.sum(le, axis=1) - 1.0).astype(jnp.int32)


def _routing(inp3d):
    return pl.pallas_call(
        _routing_body,
        out_shape=[jax.ShapeDtypeStruct((NBLK, TILE), jnp.int32),
                   jax.ShapeDtypeStruct((NT,), jnp.int32),
                   jax.ShapeDtypeStruct((NBLK, TILE, DIN), jnp.float32)],
    )(inp3d)


@functools.lru_cache(maxsize=None)
def _sc_kernels():
    mesh = plsc.VectorSubcoreMesh(core_axis_name="c", subcore_axis_name="s", num_cores=1)

    @functools.partial(
        pl.kernel,
        out_type=jax.ShapeDtypeStruct((NPAD, DIN), jnp.float32),
        mesh=mesh,
        scratch_types=[pltpu.VMEM((RPW,), jnp.int32),
                       pltpu.VMEM((RPW, DIN), jnp.float32),
                       pltpu.SemaphoreType.DMA],
    )
    def sc_scatter(x_hbm, pos_hbm, xpad_hbm, idx_v, rows_v, sem):
        wid = lax.axis_index("s") * mesh.num_cores + lax.axis_index("c")
        base = wid * RPW
        pltpu.sync_copy(x_hbm.at[pl.ds(base, RPW)], rows_v)
        pltpu.sync_copy(pos_hbm.at[pl.ds(base, RPW)], idx_v)
        pltpu.async_copy(rows_v, xpad_hbm.at[idx_v], sem).wait()

    @functools.partial(
        pl.kernel,
        out_type=jax.ShapeDtypeStruct((NTOK, DIN), jnp.float32),
        mesh=mesh,
        scratch_types=[pltpu.VMEM((RPW,), jnp.int32),
                       pltpu.VMEM((RPW, DIN), jnp.float32),
                       pltpu.SemaphoreType.DMA],
    )
    def sc_gather(ypad_hbm, pos_hbm, out_hbm, idx_v, rows_v, sem):
        wid = lax.axis_index("s") * mesh.num_cores + lax.axis_index("c")
        base = wid * RPW
        pltpu.sync_copy(pos_hbm.at[pl.ds(base, RPW)], idx_v)
        pltpu.async_copy(ypad_hbm.at[idx_v], rows_v, sem).wait()
        pltpu.sync_copy(rows_v, out_hbm.at[pl.ds(base, RPW)])

    return sc_scatter, sc_gather


KT = 20
NSTEP = NT // KT


def _mlp_body(te_ref, x_ref, w1, b1, w2, b2, w3, b3, w4, b4, w5, b5,
              w6, b6, w7, b7, y_ref):
    i = pl.program_id(0)
    tes = [te_ref[KT * i + j] for j in range(KT)]

    def lay(x, wr, br, te_j, relu):
        w = wr[te_j]
        y = lax.dot_general(x, w, (((1,), (1,)), ((), ())),
                            preferred_element_type=jnp.float32) + br[te_j]
        return jnp.maximum(y, 0.0) if relu else y

    hs = [x_ref[pl.ds(j * TILE, TILE), :] for j in range(KT)]
    for (wr, br, relu) in ((w1, b1, True), (w2, b2, True), (w3, b3, False),
                           (w4, b4, True), (w5, b5, True), (w6, b6, False),
                           (w7, b7, False)):
        hs = [lay(hs[j], wr, br, tes[j], relu) for j in range(KT)]
    for j in range(KT):
        y_ref[pl.ds(j * TILE, TILE), :] = jnp.concatenate(
            [hs[j], jnp.zeros((TILE, DIN - ACTS), jnp.float32)], axis=1)


def _mlp(texp, xpad, W1, b1, W2, b2, W3, b3, W4, b4, W5, b5, W6, b6, W7, b7):
    def wspec(out_dim):
        return pl.BlockSpec((NE, out_dim, DIN), lambda i, te: (0, 0, 0))

    def bspec(out_dim):
        return pl.BlockSpec((NE, 1, out_dim), lambda i, te: (0, 0, 0))

    in_specs = [pl.BlockSpec((KT * TILE, DIN), lambda i, te: (i, 0)),
                wspec(128), bspec(128), wspec(128), bspec(128),
                wspec(128), bspec(128), wspec(128), bspec(128),
                wspec(128), bspec(128), wspec(128), bspec(128),
                wspec(ACTS), bspec(ACTS)]

    grid_spec = pltpu.PrefetchScalarGridSpec(
        num_scalar_prefetch=1,
        grid=(NSTEP,),
        in_specs=in_specs,
        out_specs=pl.BlockSpec((KT * TILE, DIN), lambda i, te: (i, 0)),
    )
    return pl.pallas_call(
        _mlp_body,
        grid_spec=grid_spec,
        out_shape=jax.ShapeDtypeStruct((NPAD, DIN), jnp.float32),
    )(texp, xpad,
      W1, b1.reshape(NE, 1, -1), W2, b2.reshape(NE, 1, -1),
      W3, b3.reshape(NE, 1, -1), W4, b4.reshape(NE, 1, -1),
      W5, b5.reshape(NE, 1, -1), W6, b6.reshape(NE, 1, -1),
      W7, b7.reshape(NE, 1, -1))


def kernel(input, W1, b1, W2, b2, W3, b3, W4, b4, W5, b5, W6, b6, W7, b7):
    sc_scatter, sc_gather = _sc_kernels()
    pos2d, texp, x3d = _routing(input.reshape(NBLK, TILE, DIN + ID_LEN))
    pos = pos2d.reshape(NTOK)
    x = x3d.reshape(NTOK, DIN)
    xpad = sc_scatter(x, pos)
    ypad = _mlp(texp, xpad, W1, b1, W2, b2, W3, b3, W4, b4, W5, b5,
                W6, b6, W7, b7)
    out = sc_gather(ypad, pos)
    return out[:, :ACTS].reshape(NB, NA, ACTS)
